# baseline (speedup 1.0000x reference)
import math

import jax
import jax.numpy as jnp
from jax import lax
from jax.experimental import pallas as pl
from jax.experimental.pallas import tpu as pltpu
from jax.experimental.pallas import tpu_sc as plsc

_DIM = 1024
_SCALE = math.sqrt(_DIM)

_NC = 2
_NS = 16
_NW = _NC * _NS
_LANES = 16

_CHUNK = 8
_NI = 4
_NO = 4


def _scale_chunk(src, dst):
    def row_body(r, acc):
        for j in range(_DIM // _LANES):
            sl = pl.ds(j * _LANES, _LANES)
            dst[r, sl] = src[r, sl] * _SCALE
        return acc

    lax.fori_loop(0, _CHUNK, row_body, 0, unroll=False)


def _emb_kernel(table_hbm, idx_hbm, out_hbm, idx_v,
                i0, i1, i2, i3, o0, o1, o2, o3,
                g0, g1, g2, g3, s0, s1, s2, s3):
    n_chunks = idx_hbm.shape[1]
    b_per_w = n_chunks * _CHUNK
    wid = lax.axis_index("s") * _NC + lax.axis_index("c")
    base = wid * b_per_w

    ins = (i0, i1, i2, i3)
    outs = (o0, o1, o2, o3)
    gsems = (g0, g1, g2, g3)
    ssems = (s0, s1, s2, s3)

    def start_g(b, c):
        pltpu.async_copy(table_hbm.at[idx_v.at[c]], ins[b], gsems[b])

    def wait_g(b):
        pltpu.make_async_copy(table_hbm.at[idx_v.at[0]], ins[b],
                              gsems[b]).wait()

    def start_s(b, c):
        pltpu.async_copy(outs[b], out_hbm.at[pl.ds(base + c * _CHUNK, _CHUNK)],
                         ssems[b])

    def wait_s(b, c):
        pltpu.make_async_copy(
            outs[b], out_hbm.at[pl.ds(base + c * _CHUNK, _CHUNK)],
            ssems[b]).wait()

    pltpu.sync_copy(idx_hbm.at[wid], idx_v)

    for b in range(_NI):
        start_g(b, b)

    for c in range(_NI):
        wait_g(c % _NI)
        if c >= _NO:
            wait_s(c % _NO, c - _NO)
        _scale_chunk(ins[c % _NI], outs[c % _NO])
        start_g(c % _NI, c + _NI)
        start_s(c % _NO, c)

    def body(i, carry):
        for k in range(_NI):
            c = _NI * i + k
            wait_g(k)
            wait_s(k % _NO, c - _NO)
            _scale_chunk(ins[k], outs[k % _NO])
            start_g(k, c + _NI)
            start_s(k % _NO, c)
        return carry

    lax.fori_loop(1, n_chunks // _NI - 1, body, 0, unroll=2)

    for k in range(_NI):
        c = n_chunks - _NI + k
        wait_g(k)
        wait_s(c % _NO, c - _NO)
        _scale_chunk(ins[k], outs[c % _NO])
        start_s(c % _NO, c)

    for b in range(_NO):
        wait_s(b, n_chunks - _NO + b)


def kernel(x, table):
    orig_shape = x.shape
    b = x.size
    assert b % (_NW * _CHUNK) == 0
    n_chunks = b // (_NW * _CHUNK)
    idx = x.reshape(_NW, n_chunks, _CHUNK).astype(jnp.int32)

    mesh = plsc.VectorSubcoreMesh(core_axis_name="c", subcore_axis_name="s")
    run = pl.kernel(
        _emb_kernel,
        out_type=jax.ShapeDtypeStruct((b, _DIM), jnp.float32),
        mesh=mesh,
        scratch_types=(
            [pltpu.VMEM((n_chunks, _CHUNK), jnp.int32)]
            + [pltpu.VMEM((_CHUNK, _DIM), jnp.float32)] * (_NI + _NO)
            + [pltpu.SemaphoreType.DMA] * (_NI + _NO)
        ),
    )
    out = run(table, idx)
    return out.reshape(*orig_shape, _DIM)


# speedup vs baseline: 1.0211x; 1.0211x over previous
import math

import jax
import jax.numpy as jnp
from jax import lax
from jax.experimental import pallas as pl
from jax.experimental.pallas import tpu as pltpu
from jax.experimental.pallas import tpu_sc as plsc

_DIM = 1024
_SCALE = math.sqrt(_DIM)

_NC = 2
_NS = 16
_NW = _NC * _NS
_LANES = 16

_CHUNK = 8
_NI = 4
_NO = 4


def _scale_chunk(src, dst):
    def row_body(r, acc):
        for j in range(_DIM // _LANES):
            sl = pl.ds(j * _LANES, _LANES)
            dst[r, sl] = src[r, sl] * _SCALE
        return acc

    lax.fori_loop(0, _CHUNK, row_body, 0, unroll=False)


def _emb_kernel(table_hbm, idx_hbm, out_hbm, idx_v,
                i0, i1, i2, i3, o0, o1, o2, o3,
                g0, g1, g2, g3, s0, s1, s2, s3):
    n_chunks = idx_hbm.shape[1]
    b_per_w = n_chunks * _CHUNK
    wid = lax.axis_index("s") * _NC + lax.axis_index("c")
    base = wid * b_per_w

    ins = (i0, i1, i2, i3)
    outs = (o0, o1, o2, o3)
    gsems = (g0, g1, g2, g3)
    ssems = (s0, s1, s2, s3)

    def start_g(b, c):
        pltpu.async_copy(table_hbm.at[idx_v.at[c]], ins[b], gsems[b])

    def wait_g(b):
        pltpu.make_async_copy(table_hbm.at[idx_v.at[0]], ins[b],
                              gsems[b]).wait()

    def start_s(b, c):
        pltpu.async_copy(outs[b], out_hbm.at[pl.ds(base + c * _CHUNK, _CHUNK)],
                         ssems[b])

    def wait_s(b, c):
        pltpu.make_async_copy(
            outs[b], out_hbm.at[pl.ds(base + c * _CHUNK, _CHUNK)],
            ssems[b]).wait()

    pltpu.sync_copy(idx_hbm.at[wid], idx_v)

    for b in range(_NI):
        start_g(b, b)

    for c in range(_NI):
        wait_g(c % _NI)
        if c >= _NO:
            wait_s(c % _NO, c - _NO)
        _scale_chunk(ins[c % _NI], outs[c % _NO])
        start_g(c % _NI, c + _NI)
        start_s(c % _NO, c)

    def body(i, carry):
        for k in range(_NI):
            c = _NI * i + k
            wait_g(k)
            wait_s(k % _NO, c - _NO)
            _scale_chunk(ins[k], outs[k % _NO])
            start_g(k, c + _NI)
            start_s(k % _NO, c)
        return carry

    lax.fori_loop(1, n_chunks // _NI - 1, body, 0, unroll=False)

    for k in range(_NI):
        c = n_chunks - _NI + k
        wait_g(k)
        wait_s(c % _NO, c - _NO)
        _scale_chunk(ins[k], outs[c % _NO])
        start_s(c % _NO, c)

    for b in range(_NO):
        wait_s(b, n_chunks - _NO + b)


def kernel(x, table):
    orig_shape = x.shape
    b = x.size
    assert b % (_NW * _CHUNK) == 0
    n_chunks = b // (_NW * _CHUNK)
    idx = x.reshape(_NW, n_chunks, _CHUNK).astype(jnp.int32)

    mesh = plsc.VectorSubcoreMesh(core_axis_name="c", subcore_axis_name="s")
    run = pl.kernel(
        _emb_kernel,
        out_type=jax.ShapeDtypeStruct((b, _DIM), jnp.float32),
        mesh=mesh,
        scratch_types=(
            [pltpu.VMEM((n_chunks, _CHUNK), jnp.int32)]
            + [pltpu.VMEM((_CHUNK, _DIM), jnp.float32)] * (_NI + _NO)
            + [pltpu.SemaphoreType.DMA] * (_NI + _NO)
        ),
    )
    out = run(table, idx)
    return out.reshape(*orig_shape, _DIM)
